# baseline (speedup 1.0000x reference)
import functools

import jax
import jax.numpy as jnp
from jax import lax
from jax.experimental import pallas as pl
from jax.experimental.pallas import tpu as pltpu
from jax.experimental.pallas import tpu_sc as plsc

NC = 2
NS = 16
L = 16
NW = NC * NS
RPT = 320
NPAD = RPT * NW
KB = 80
EB = 2 * KB
CHUNK = 4000
N_EDGES = 320000
CAPB = N_EDGES // KB + 1

_SC_PARAMS = pltpu.CompilerParams(needs_layout_passes=False)


def _mesh():
    return plsc.VectorSubcoreMesh(
        core_axis_name="c", subcore_axis_name="s", num_cores=NC, num_subcores=NS
    )


def _init_acc(acc, C):
    zero16f = jnp.zeros((L,), jnp.float32)

    def init_row(r, carry):
        for c in range(C // L):
            acc[r, pl.ds(c * L, L)] = zero16f
        return carry

    lax.fori_loop(0, RPT + 1, init_row, 0)


def _reset_block(ebuf, off):
    z = jnp.zeros((L,), jnp.int32)
    t = jnp.full((L,), RPT, jnp.int32)
    for i in range(KB // L):
        ebuf[pl.ds(off + i * L, L)] = z
        ebuf[pl.ds(off + KB + i * L, L)] = t


def _process_batch(acc, gbuf, ebuf, goff, doff, C):
    def group(g, carry):
        d16 = ebuf[pl.ds(doff + g * L, L)]
        for j in range(L):
            dj = d16[j]
            row = goff + g * L + j
            for c in range(C // L):
                cs = c * L
                acc[dj, pl.ds(cs, L)] = jnp.maximum(
                    acc[dj, pl.ds(cs, L)], gbuf[row, pl.ds(cs, L)]
                )
        return carry

    lax.fori_loop(0, KB // L, group, 0)


def _scan_scatter_max(hp, src, dst, C):
    nchunks = N_EDGES // CHUNK

    def body(hp_ref, src_ref, dst_ref, out_ref, el_ref, nb_ref,
             acc, gbuf, ebuf, dch, sch, nbv,
             gsem0, gsem1, wsem0, wsem1, csem0, csem1):
        wid = lax.axis_index("s") * NC + lax.axis_index("c")
        lo = wid * RPT

        _init_acc(acc, C)
        _reset_block(ebuf, 0)
        _reset_block(ebuf, EB)

        gsems = (gsem0, gsem1)
        wsems = (wsem0, wsem1)

        el_base = wid * (CAPB * EB)

        def issue_batch(P, nb):
            o = P * EB
            pltpu.async_copy(ebuf.at[pl.ds(o, EB)],
                             el_ref.at[pl.ds(el_base + nb * EB, EB)], wsems[P])
            pltpu.async_copy(hp_ref.at[ebuf.at[pl.ds(o, KB)]],
                             gbuf.at[pl.ds(P * KB, KB)], gsems[P])

        def drain_batch(P):
            o = P * EB
            pltpu.make_async_copy(hp_ref.at[ebuf.at[pl.ds(o, KB)]],
                                  gbuf.at[pl.ds(P * KB, KB)], gsems[P]).wait()
            _process_batch(acc, gbuf, ebuf, P * KB, o + KB, C)
            pltpu.make_async_copy(ebuf.at[pl.ds(o, EB)],
                                  el_ref.at[pl.ds(0, EB)], wsems[P]).wait()
            _reset_block(ebuf, o)

        def flush(carry, P):
            cnt, p, primed, nb = carry
            issue_batch(P, nb)

            @pl.when(primed == 1)
            def _():
                drain_batch(1 - P)

            return (jnp.int32(0), jnp.int32(1 - P), jnp.int32(1), nb + 1)

        def step(i, carry):
            cnt, p, primed, nb = carry
            d = dch[pl.ds(i * L, L)]
            s = sch[pl.ds(i * L, L)]
            dl = d - lo
            m = (dl >= 0) & (dl < RPT)
            base = p * EB + cnt
            plsc.store_compressed(ebuf.at[pl.ds(base, L)], s, mask=m)
            plsc.store_compressed(ebuf.at[pl.ds(base + KB, L)], dl, mask=m)
            cnt = cnt + plsc.all_reduce_population_count(m)[0]
            carry = (cnt, p, primed, nb)
            do_flush = cnt > KB - L
            carry = lax.cond(do_flush & (p == 0),
                             lambda c: flush(c, 0), lambda c: c, carry)
            carry = lax.cond(do_flush & (p == 1),
                             lambda c: flush(c, 1), lambda c: c, carry)
            return carry

        csems = (csem0, csem1)

        def issue_chunk(ci, B):
            off = B * CHUNK
            pltpu.async_copy(dst_ref.at[pl.ds(ci * CHUNK, CHUNK)],
                             dch.at[pl.ds(off, CHUNK)], csems[B])
            pltpu.async_copy(src_ref.at[pl.ds(ci * CHUNK, CHUNK)],
                             sch.at[pl.ds(off, CHUNK)], csems[B])

        def wait_chunk(B):
            off = B * CHUNK
            pltpu.make_async_copy(dst_ref.at[pl.ds(0, CHUNK)],
                                  dch.at[pl.ds(off, CHUNK)], csems[B]).wait()
            pltpu.make_async_copy(src_ref.at[pl.ds(0, CHUNK)],
                                  sch.at[pl.ds(off, CHUNK)], csems[B]).wait()

        issue_chunk(0, 0)
        issue_chunk(1, 1)

        def chunk_pair(k, carry):
            def scan_buf(B, ci, carry):
                wait_chunk(B)

                def inner(i, c):
                    return step(B * (CHUNK // L) + i, c)

                carry = lax.fori_loop(0, CHUNK // L, inner, carry)

                @pl.when(ci + 2 < nchunks)
                def _():
                    issue_chunk(ci + 2, B)

                return carry

            carry = scan_buf(0, 2 * k, carry)
            carry = scan_buf(1, 2 * k + 1, carry)
            return carry

        carry = lax.fori_loop(0, nchunks // 2, chunk_pair,
                              (jnp.int32(0), jnp.int32(0), jnp.int32(0),
                               jnp.int32(0)))

        def fin(carry, P):
            cnt, p, primed, nb = carry
            issue_batch(P, nb)

            @pl.when(primed == 1)
            def _():
                drain_batch(1 - P)

            drain_batch(P)
            return nb + 1

        nb_final = lax.cond(carry[1] == 0,
                            lambda c: fin(c, 0), lambda c: fin(c, 1), carry)

        nbv[pl.ds(0, L)] = jnp.full((L,), nb_final, jnp.int32)
        pltpu.sync_copy(nbv, nb_ref.at[pl.ds(wid * L, L)])
        pltpu.sync_copy(acc.at[pl.ds(0, RPT)], out_ref.at[pl.ds(wid * RPT, RPT)])

    kern = pl.kernel(
        body,
        out_type=(
            jax.ShapeDtypeStruct((NPAD, C), jnp.float32),
            jax.ShapeDtypeStruct((NW * CAPB * EB,), jnp.int32),
            jax.ShapeDtypeStruct((NW * L,), jnp.int32),
        ),
        mesh=_mesh(),
        scratch_types=[
            pltpu.VMEM((RPT + 1, C), jnp.float32),
            pltpu.VMEM((2 * KB, C), jnp.float32),
            pltpu.VMEM((2 * EB,), jnp.int32),
            pltpu.VMEM((2 * CHUNK,), jnp.int32),
            pltpu.VMEM((2 * CHUNK,), jnp.int32),
            pltpu.VMEM((L,), jnp.int32),
            pltpu.SemaphoreType.DMA,
            pltpu.SemaphoreType.DMA,
            pltpu.SemaphoreType.DMA,
            pltpu.SemaphoreType.DMA,
            pltpu.SemaphoreType.DMA,
            pltpu.SemaphoreType.DMA,
        ],
        compiler_params=_SC_PARAMS,
    )
    return kern(hp, src, dst)


def _replay_scatter_max(hp, el, nb, C):
    def body(hp_ref, el_ref, nb_ref, out_ref,
             acc, gbuf, ebuf, nbv, gsem0, gsem1):
        wid = lax.axis_index("s") * NC + lax.axis_index("c")

        _init_acc(acc, C)

        pltpu.sync_copy(nb_ref.at[pl.ds(wid * L, L)], nbv)
        nb = nbv[pl.ds(0, L)][0]

        gsems = (gsem0, gsem1)
        el_base = wid * (CAPB * EB)

        def load_idx(b, P):
            pltpu.sync_copy(el_ref.at[pl.ds(el_base + b * EB, EB)],
                            ebuf.at[pl.ds(P * EB, EB)])

        def issue_gather(P):
            pltpu.async_copy(hp_ref.at[ebuf.at[pl.ds(P * EB, KB)]],
                             gbuf.at[pl.ds(P * KB, KB)], gsems[P])

        def wait_gather(P):
            pltpu.make_async_copy(hp_ref.at[ebuf.at[pl.ds(P * EB, KB)]],
                                  gbuf.at[pl.ds(P * KB, KB)], gsems[P]).wait()

        @pl.when(0 < nb)
        def _():
            load_idx(jnp.int32(0), 0)
            issue_gather(0)

        @pl.when(1 < nb)
        def _():
            load_idx(jnp.int32(1), 1)
            issue_gather(1)

        def pair(k, carry):
            for P in range(2):
                b = 2 * k + P

                @pl.when(b < nb)
                def _():
                    wait_gather(P)
                    _process_batch(acc, gbuf, ebuf, P * KB, P * EB + KB, C)

                    @pl.when(b + 2 < nb)
                    def _():
                        load_idx(b + 2, P)
                        issue_gather(P)

            return carry

        lax.fori_loop(0, (nb + 1) // 2, pair, 0)
        pltpu.sync_copy(acc.at[pl.ds(0, RPT)], out_ref.at[pl.ds(wid * RPT, RPT)])

    kern = pl.kernel(
        body,
        out_type=jax.ShapeDtypeStruct((NPAD, C), jnp.float32),
        mesh=_mesh(),
        scratch_types=[
            pltpu.VMEM((RPT + 1, C), jnp.float32),
            pltpu.VMEM((2 * KB, C), jnp.float32),
            pltpu.VMEM((2 * EB,), jnp.int32),
            pltpu.VMEM((L,), jnp.int32),
            pltpu.SemaphoreType.DMA,
            pltpu.SemaphoreType.DMA,
        ],
        compiler_params=_SC_PARAMS,
    )
    return kern(hp, el, nb)


def _dot_t(x, w):
    return lax.dot_general(x, w, (((1,), (1,)), ((), ())),
                           preferred_element_type=jnp.float32)


def _pool_body(x_ref, w_ref, b_ref, o_ref):
    o_ref[...] = jnp.maximum(_dot_t(x_ref[...], w_ref[...]) + b_ref[...], 0.0)


def _mid_body(a_ref, wl_ref, bl_ref, wp_ref, bp_ref, o_ref):
    t = jnp.maximum(_dot_t(a_ref[...], wl_ref[...]) + bl_ref[...], 0.0)
    norm = jnp.sqrt(jnp.sum(t * t, axis=1, keepdims=True))
    h = t / jnp.maximum(norm, 1e-12)
    o_ref[...] = jnp.maximum(_dot_t(h, wp_ref[...]) + bp_ref[...], 0.0)


def _final_body(a_ref, w_ref, b_ref, o_ref):
    o_ref[...] = _dot_t(a_ref[...], w_ref[...]) + b_ref[...]


def _tc_call(body, n_in, x, *weights, R):
    M, K = x.shape
    N = weights[-2].shape[0] if n_in > 3 else weights[0].shape[0]
    specs = [pl.BlockSpec((R, K), lambda i: (i, 0))]
    for w in weights:
        shp = w.shape if w.ndim == 2 else (1, w.shape[0])
        specs.append(pl.BlockSpec(shp, lambda i: (0, 0)))
    args = [x] + [w if w.ndim == 2 else w.reshape(1, -1) for w in weights]
    return pl.pallas_call(
        body,
        grid=(M // R,),
        in_specs=specs,
        out_specs=pl.BlockSpec((R, N), lambda i: (i, 0)),
        out_shape=jax.ShapeDtypeStruct((M, N), jnp.float32),
    )(*args)


def kernel(node_feats, edge_index, W0p, b0p, W0, b0, bias0,
           W1p, b1p, W1, b1, bias1, W2p, b2p, W2, b2, bias2):
    src = edge_index[0].astype(jnp.int32)
    dst = edge_index[1].astype(jnp.int32)

    hp0 = _tc_call(_pool_body, 3, node_feats, W0p, b0p, R=1000)
    agg0, el, nb = _scan_scatter_max(hp0, src, dst, 128)
    hp1 = _tc_call(_mid_body, 5, agg0, W0, b0 + bias0, W1p, b1p, R=1024)
    agg1 = _replay_scatter_max(hp1, el, nb, 256)
    hp2 = _tc_call(_mid_body, 5, agg1, W1, b1 + bias1, W2p, b2p, R=1024)
    agg2 = _replay_scatter_max(hp2, el, nb, 256)
    w2pad = jnp.zeros((128, 256), jnp.float32).at[:64].set(W2)
    b2pad = jnp.zeros((128,), jnp.float32).at[:64].set(b2 + bias2)
    out = _tc_call(_final_body, 3, agg2, w2pad, b2pad, R=1024)
    return out[:10000, :64]


# speedup vs baseline: 1.5809x; 1.5809x over previous
import functools

import jax
import jax.numpy as jnp
from jax import lax
from jax.experimental import pallas as pl
from jax.experimental.pallas import tpu as pltpu
from jax.experimental.pallas import tpu_sc as plsc

NC = 2
NS = 16
L = 16
NW = NC * NS
RPT = 320
NPAD = RPT * NW
KB = 128
CHUNK = 3200
N_EDGES = 320000

_SC_PARAMS = pltpu.CompilerParams(needs_layout_passes=False)


def _scatter_max(hp, src, dst, C):
    nchunks = N_EDGES // CHUNK

    def body(hp_ref, src_ref, dst_ref, out_ref,
             acc, gbuf, srcb, dstb, dch, sch, gsem, csem0, csem1):
        wid = lax.axis_index("s") * NC + lax.axis_index("c")
        lo = wid * RPT

        zero16f = jnp.zeros((L,), jnp.float32)
        trash16 = jnp.full((L,), RPT, jnp.int32)
        zero16i = jnp.zeros((L,), jnp.int32)

        def init_row(r, carry):
            for c in range(C // L):
                acc[r, pl.ds(c * L, L)] = zero16f
            return carry

        lax.fori_loop(0, RPT + 1, init_row, 0)

        def reset_batch():
            for i in range(KB // L):
                dstb[pl.ds(i * L, L)] = trash16
                srcb[pl.ds(i * L, L)] = zero16i

        reset_batch()

        def flush():
            pltpu.async_copy(hp_ref.at[srcb], gbuf, gsem).wait()

            def group(g, carry):
                d16 = dstb[pl.ds(g * L, L)]
                for j in range(L):
                    dj = d16[j]
                    e = g * L + j
                    for c in range(C // L):
                        cs = c * L
                        acc[dj, pl.ds(cs, L)] = jnp.maximum(
                            acc[dj, pl.ds(cs, L)], gbuf[e, pl.ds(cs, L)]
                        )
                return carry

            lax.fori_loop(0, KB // L, group, 0)
            reset_batch()

        def step(i, cnt):
            d = dch[pl.ds(i * L, L)]
            s = sch[pl.ds(i * L, L)]
            dl = d - lo
            m = (dl >= 0) & (dl < RPT)
            plsc.store_compressed(dstb.at[pl.ds(cnt, L)], dl, mask=m)
            plsc.store_compressed(srcb.at[pl.ds(cnt, L)], s, mask=m)
            cnt = cnt + plsc.all_reduce_population_count(m)[0]

            def do_flush():
                flush()
                return jnp.int32(0)

            return lax.cond(cnt > KB - L, do_flush, lambda: cnt)

        csems = (csem0, csem1)

        def issue_chunk(ci, B):
            off = B * CHUNK
            pltpu.async_copy(dst_ref.at[pl.ds(ci * CHUNK, CHUNK)],
                             dch.at[pl.ds(off, CHUNK)], csems[B])
            pltpu.async_copy(src_ref.at[pl.ds(ci * CHUNK, CHUNK)],
                             sch.at[pl.ds(off, CHUNK)], csems[B])

        def wait_chunk(B):
            off = B * CHUNK
            pltpu.make_async_copy(dst_ref.at[pl.ds(0, CHUNK)],
                                  dch.at[pl.ds(off, CHUNK)], csems[B]).wait()
            pltpu.make_async_copy(src_ref.at[pl.ds(0, CHUNK)],
                                  sch.at[pl.ds(off, CHUNK)], csems[B]).wait()

        issue_chunk(0, 0)
        issue_chunk(1, 1)

        def chunk_pair(k, cnt):
            def scan_buf(B, ci, cnt):
                wait_chunk(B)

                def inner(i, c):
                    return step(B * (CHUNK // L) + i, c)

                cnt = lax.fori_loop(0, CHUNK // L, inner, cnt)

                @pl.when(ci + 2 < nchunks)
                def _():
                    issue_chunk(ci + 2, B)

                return cnt

            cnt = scan_buf(0, 2 * k, cnt)
            cnt = scan_buf(1, 2 * k + 1, cnt)
            return cnt

        lax.fori_loop(0, nchunks // 2, chunk_pair, jnp.int32(0))
        flush()
        pltpu.sync_copy(acc.at[pl.ds(0, RPT)], out_ref.at[pl.ds(lo, RPT)])

    kern = pl.kernel(
        body,
        out_type=jax.ShapeDtypeStruct((NPAD, C), jnp.float32),
        mesh=plsc.VectorSubcoreMesh(core_axis_name="c", subcore_axis_name="s",
                                    num_cores=NC, num_subcores=NS),
        scratch_types=[
            pltpu.VMEM((RPT + 1, C), jnp.float32),
            pltpu.VMEM((KB, C), jnp.float32),
            pltpu.VMEM((KB,), jnp.int32),
            pltpu.VMEM((KB,), jnp.int32),
            pltpu.VMEM((2 * CHUNK,), jnp.int32),
            pltpu.VMEM((2 * CHUNK,), jnp.int32),
            pltpu.SemaphoreType.DMA,
            pltpu.SemaphoreType.DMA,
            pltpu.SemaphoreType.DMA,
        ],
        compiler_params=_SC_PARAMS,
    )
    return kern(hp, src, dst)


def _dot_t(x, w):
    return lax.dot_general(x, w, (((1,), (1,)), ((), ())),
                           preferred_element_type=jnp.float32)


def _pool_body(x_ref, w_ref, b_ref, o_ref):
    o_ref[...] = jnp.maximum(_dot_t(x_ref[...], w_ref[...]) + b_ref[...], 0.0)


def _mid_body(a_ref, wl_ref, bl_ref, wp_ref, bp_ref, o_ref):
    t = jnp.maximum(_dot_t(a_ref[...], wl_ref[...]) + bl_ref[...], 0.0)
    norm = jnp.sqrt(jnp.sum(t * t, axis=1, keepdims=True))
    h = t / jnp.maximum(norm, 1e-12)
    o_ref[...] = jnp.maximum(_dot_t(h, wp_ref[...]) + bp_ref[...], 0.0)


def _final_body(a_ref, w_ref, b_ref, o_ref):
    o_ref[...] = _dot_t(a_ref[...], w_ref[...]) + b_ref[...]


def _tc_call(body, n_in, x, *weights, R):
    M, K = x.shape
    N = weights[-2].shape[0] if n_in > 3 else weights[0].shape[0]
    specs = [pl.BlockSpec((R, K), lambda i: (i, 0))]
    for w in weights:
        shp = w.shape if w.ndim == 2 else (1, w.shape[0])
        specs.append(pl.BlockSpec(shp, lambda i: (0, 0)))
    args = [x] + [w if w.ndim == 2 else w.reshape(1, -1) for w in weights]
    return pl.pallas_call(
        body,
        grid=(M // R,),
        in_specs=specs,
        out_specs=pl.BlockSpec((R, N), lambda i: (i, 0)),
        out_shape=jax.ShapeDtypeStruct((M, N), jnp.float32),
    )(*args)


def kernel(node_feats, edge_index, W0p, b0p, W0, b0, bias0,
           W1p, b1p, W1, b1, bias1, W2p, b2p, W2, b2, bias2):
    src = edge_index[0].astype(jnp.int32)
    dst = edge_index[1].astype(jnp.int32)

    hp0 = _tc_call(_pool_body, 3, node_feats, W0p, b0p, R=1000)
    agg0 = _scatter_max(hp0, src, dst, 128)
    hp1 = _tc_call(_mid_body, 5, agg0, W0, b0 + bias0, W1p, b1p, R=1024)
    agg1 = _scatter_max(hp1, src, dst, 256)
    hp2 = _tc_call(_mid_body, 5, agg1, W1, b1 + bias1, W2p, b2p, R=1024)
    agg2 = _scatter_max(hp2, src, dst, 256)
    w2pad = jnp.zeros((128, 256), jnp.float32).at[:64].set(W2)
    b2pad = jnp.zeros((128,), jnp.float32).at[:64].set(b2 + bias2)
    out = _tc_call(_final_body, 3, agg2, w2pad, b2pad, R=1024)
    return out[:10000, :64]


# speedup vs baseline: 1.5969x; 1.0101x over previous
import functools

import jax
import jax.numpy as jnp
from jax import lax
from jax.experimental import pallas as pl
from jax.experimental.pallas import tpu as pltpu
from jax.experimental.pallas import tpu_sc as plsc

NC = 2
NS = 16
L = 16
NW = NC * NS
RPT = 320
NPAD = RPT * NW
KB = 128
CHUNK = 3200
N_EDGES = 320000

_SC_PARAMS = pltpu.CompilerParams(needs_layout_passes=False)


def _scatter_max(hp, src, dst, C, packed=False):
    nchunks = N_EDGES // CHUNK
    GW = C // 2 if packed else C

    def body(hp_ref, src_ref, dst_ref, out_ref,
             acc, gbuf, srcb, dstb, dch, sch, gsem, csem0, csem1):
        wid = lax.axis_index("s") * NC + lax.axis_index("c")
        lo = wid * RPT

        zero16f = jnp.zeros((L,), jnp.float32)
        trash16 = jnp.full((L,), RPT, jnp.int32)
        zero16i = jnp.zeros((L,), jnp.int32)

        def init_row(r, carry):
            for c in range(C // L):
                acc[r, pl.ds(c * L, L)] = zero16f
            return carry

        lax.fori_loop(0, RPT + 1, init_row, 0)

        def reset_batch():
            for i in range(KB // L):
                dstb[pl.ds(i * L, L)] = trash16
                srcb[pl.ds(i * L, L)] = zero16i

        reset_batch()

        def flush():
            pltpu.async_copy(hp_ref.at[srcb], gbuf, gsem).wait()

            def group(g, carry):
                d16 = dstb[pl.ds(g * L, L)]
                for j in range(L):
                    dj = d16[j]
                    e = g * L + j
                    if packed:
                        for c in range(C // (2 * L)):
                            w = gbuf[e, pl.ds(c * L, L)]
                            v = plsc.bitcast(w, jnp.bfloat16)
                            a, b = plsc.unpack(
                                v, format=plsc.PackFormat.INTERLEAVED,
                                preferred_element_type=jnp.float32)
                            ca = c * 2 * L
                            acc[dj, pl.ds(ca, L)] = jnp.maximum(
                                acc[dj, pl.ds(ca, L)], a)
                            acc[dj, pl.ds(ca + L, L)] = jnp.maximum(
                                acc[dj, pl.ds(ca + L, L)], b)
                    else:
                        for c in range(C // L):
                            cs = c * L
                            acc[dj, pl.ds(cs, L)] = jnp.maximum(
                                acc[dj, pl.ds(cs, L)], gbuf[e, pl.ds(cs, L)]
                            )
                return carry

            lax.fori_loop(0, KB // L, group, 0)
            reset_batch()

        def step(i, cnt):
            d = dch[pl.ds(i * L, L)]
            s = sch[pl.ds(i * L, L)]
            dl = d - lo
            m = (dl >= 0) & (dl < RPT)
            plsc.store_compressed(dstb.at[pl.ds(cnt, L)], dl, mask=m)
            plsc.store_compressed(srcb.at[pl.ds(cnt, L)], s, mask=m)
            cnt = cnt + plsc.all_reduce_population_count(m)[0]

            def do_flush():
                flush()
                return jnp.int32(0)

            return lax.cond(cnt > KB - L, do_flush, lambda: cnt)

        csems = (csem0, csem1)

        def issue_chunk(ci, B):
            off = B * CHUNK
            pltpu.async_copy(dst_ref.at[pl.ds(ci * CHUNK, CHUNK)],
                             dch.at[pl.ds(off, CHUNK)], csems[B])
            pltpu.async_copy(src_ref.at[pl.ds(ci * CHUNK, CHUNK)],
                             sch.at[pl.ds(off, CHUNK)], csems[B])

        def wait_chunk(B):
            off = B * CHUNK
            pltpu.make_async_copy(dst_ref.at[pl.ds(0, CHUNK)],
                                  dch.at[pl.ds(off, CHUNK)], csems[B]).wait()
            pltpu.make_async_copy(src_ref.at[pl.ds(0, CHUNK)],
                                  sch.at[pl.ds(off, CHUNK)], csems[B]).wait()

        issue_chunk(0, 0)
        issue_chunk(1, 1)

        def chunk_pair(k, cnt):
            def scan_buf(B, ci, cnt):
                wait_chunk(B)

                def inner(i, c):
                    return step(B * (CHUNK // L) + i, c)

                cnt = lax.fori_loop(0, CHUNK // L, inner, cnt)

                @pl.when(ci + 2 < nchunks)
                def _():
                    issue_chunk(ci + 2, B)

                return cnt

            cnt = scan_buf(0, 2 * k, cnt)
            cnt = scan_buf(1, 2 * k + 1, cnt)
            return cnt

        lax.fori_loop(0, nchunks // 2, chunk_pair, jnp.int32(0))
        flush()
        pltpu.sync_copy(acc.at[pl.ds(0, RPT)], out_ref.at[pl.ds(lo, RPT)])

    kern = pl.kernel(
        body,
        out_type=jax.ShapeDtypeStruct((NPAD, C), jnp.float32),
        mesh=plsc.VectorSubcoreMesh(core_axis_name="c", subcore_axis_name="s",
                                    num_cores=NC, num_subcores=NS),
        scratch_types=[
            pltpu.VMEM((RPT + 1, C), jnp.float32),
            pltpu.VMEM((KB, GW),
                       jnp.int32 if packed else jnp.float32),
            pltpu.VMEM((KB,), jnp.int32),
            pltpu.VMEM((KB,), jnp.int32),
            pltpu.VMEM((2 * CHUNK,), jnp.int32),
            pltpu.VMEM((2 * CHUNK,), jnp.int32),
            pltpu.SemaphoreType.DMA,
            pltpu.SemaphoreType.DMA,
            pltpu.SemaphoreType.DMA,
        ],
        compiler_params=_SC_PARAMS,
    )
    return kern(hp, src, dst)


def _dot_t(x, w):
    return lax.dot_general(x, w, (((1,), (1,)), ((), ())),
                           preferred_element_type=jnp.float32)


def _pool_body(x_ref, w_ref, b_ref, o_ref):
    o_ref[...] = jnp.maximum(_dot_t(x_ref[...], w_ref[...]) + b_ref[...], 0.0)


def _mid_body(a_ref, wl_ref, bl_ref, wp_ref, bp_ref, o_ref):
    t = jnp.maximum(_dot_t(a_ref[...], wl_ref[...]) + bl_ref[...], 0.0)
    norm = jnp.sqrt(jnp.sum(t * t, axis=1, keepdims=True))
    h = t / jnp.maximum(norm, 1e-12)
    y = jnp.maximum(_dot_t(h, wp_ref[...]) + bp_ref[...], 0.0)
    o_ref[...] = y.astype(o_ref.dtype)


def _final_body(a_ref, w_ref, b_ref, o_ref):
    o_ref[...] = _dot_t(a_ref[...], w_ref[...]) + b_ref[...]


def _tc_call(body, n_in, x, *weights, R, out_dtype=jnp.float32):
    M, K = x.shape
    N = weights[-2].shape[0] if n_in > 3 else weights[0].shape[0]
    specs = [pl.BlockSpec((R, K), lambda i: (i, 0))]
    for w in weights:
        shp = w.shape if w.ndim == 2 else (1, w.shape[0])
        specs.append(pl.BlockSpec(shp, lambda i: (0, 0)))
    args = [x] + [w if w.ndim == 2 else w.reshape(1, -1) for w in weights]
    return pl.pallas_call(
        body,
        grid=(M // R,),
        in_specs=specs,
        out_specs=pl.BlockSpec((R, N), lambda i: (i, 0)),
        out_shape=jax.ShapeDtypeStruct((M, N), out_dtype),
    )(*args)


def _as_i32(hp_bf16):
    n, c = hp_bf16.shape
    return lax.bitcast_convert_type(hp_bf16.reshape(n, c // 2, 2), jnp.int32)


def _deinterleave_perm(C):
    perm = []
    for k in range(C // 32):
        perm.extend(32 * k + 2 * i for i in range(16))
        perm.extend(32 * k + 2 * i + 1 for i in range(16))
    return jnp.array(perm, dtype=jnp.int32)


def kernel(node_feats, edge_index, W0p, b0p, W0, b0, bias0,
           W1p, b1p, W1, b1, bias1, W2p, b2p, W2, b2, bias2):
    src = edge_index[0].astype(jnp.int32)
    dst = edge_index[1].astype(jnp.int32)

    perm = _deinterleave_perm(256)
    hp0 = _tc_call(_pool_body, 3, node_feats, W0p, b0p, R=1000)
    agg0 = _scatter_max(hp0, src, dst, 128)
    hp1 = _tc_call(_mid_body, 5, agg0, W0, b0 + bias0, W1p, b1p, R=1024,
                   out_dtype=jnp.bfloat16)
    agg1 = _scatter_max(_as_i32(hp1), src, dst, 256, packed=True)
    hp2 = _tc_call(_mid_body, 5, agg1, W1[:, perm], b1 + bias1, W2p, b2p,
                   R=1024, out_dtype=jnp.bfloat16)
    agg2 = _scatter_max(_as_i32(hp2), src, dst, 256, packed=True)
    w2pad = jnp.zeros((128, 256), jnp.float32).at[:64].set(W2)
    b2pad = jnp.zeros((128,), jnp.float32).at[:64].set(b2 + bias2)
    out = _tc_call(_final_body, 3, agg2, w2pad[:, perm], b2pad, R=1024)
    return out[:10000, :64]
